# baseline (speedup 1.0000x reference)
import functools

import jax
import jax.numpy as jnp
from jax import lax
from jax.experimental import pallas as pl
from jax.experimental.pallas import tpu as pltpu
from jax.experimental.pallas import tpu_sc as plsc

B = 16384
D = 64
L = 16
NC = 2
NS = 16
NW = NC * NS
BPW = B // NW
HALF = 256

_mesh = plsc.VectorSubcoreMesh(core_axis_name="c", subcore_axis_name="s")


@functools.partial(
    pl.kernel,
    mesh=_mesh,
    compiler_params=pltpu.CompilerParams(needs_layout_passes=False),
    out_type=jax.ShapeDtypeStruct((B, D), jnp.float32),
    scratch_types=[
        pltpu.VMEM((BPW,), jnp.int32),
        pltpu.VMEM((HALF, D), jnp.float32),
        pltpu.SemaphoreType.DMA,
    ],
)
def _sc_user_gather(uid_hbm, ut_hbm, out_hbm, uidx_v, urows_v, usem):
    wid = lax.axis_index("s") * NC + lax.axis_index("c")
    base = wid * BPW

    pltpu.sync_copy(uid_hbm.at[pl.ds(base, BPW)], uidx_v)

    for h in range(2):
        hbase = h * HALF
        copies = []
        for k in range(HALF):
            if k % L == 0:
                uvec = uidx_v[pl.ds(hbase + k, L)]
            copies.append(pltpu.async_copy(
                ut_hbm.at[uvec[k % L]], urows_v.at[k], usem))
        for cp in copies:
            cp.wait()
        pltpu.sync_copy(
            urows_v,
            out_hbm.at[pl.ds(pl.multiple_of(base + hbase, HALF), HALF)])


_TCROWS = 8


def _tc_gather_body(iid_ref, *refs):
    o_ref = refs[_TCROWS]
    b = pl.program_id(0)
    for j in range(_TCROWS):
        r = iid_ref[b * _TCROWS + j] % 8
        o_ref[pl.ds(j, 1), :] = refs[j][pl.ds(r, 1), :]


def _tc_item_gather(item_ids, item_table):
    grid_spec = pltpu.PrefetchScalarGridSpec(
        num_scalar_prefetch=1,
        grid=(B // _TCROWS,),
        in_specs=[
            pl.BlockSpec(
                (8, D),
                (lambda b, iid, j=j: (iid[b * _TCROWS + j] // 8, 0)))
            for j in range(_TCROWS)
        ],
        out_specs=pl.BlockSpec((_TCROWS, D), lambda b, iid: (b, 0)),
    )
    return pl.pallas_call(
        _tc_gather_body,
        grid_spec=grid_spec,
        out_shape=jax.ShapeDtypeStruct((B, D), jnp.float32),
    )(item_ids, *([item_table] * _TCROWS))


def _tc_combine_body(u_ref, i_ref, w_ref, b_ref, o_ref):
    prod = u_ref[...] * i_ref[...] * w_ref[...]
    o_ref[...] = prod.sum(axis=1) + b_ref[0, 0]


_TCBLK = 2048


def _tc_combine(uw, ir, w, b):
    return pl.pallas_call(
        _tc_combine_body,
        grid=(B // _TCBLK,),
        in_specs=[
            pl.BlockSpec((_TCBLK, D), lambda g: (g, 0)),
            pl.BlockSpec((_TCBLK, D), lambda g: (g, 0)),
            pl.BlockSpec((1, D), lambda g: (0, 0)),
            pl.BlockSpec((1, 1), lambda g: (0, 0), memory_space=pltpu.SMEM),
        ],
        out_specs=pl.BlockSpec((_TCBLK,), lambda g: (g,)),
        out_shape=jax.ShapeDtypeStruct((B,), jnp.float32),
    )(uw, ir, w, b)


def kernel(user_ids, item_ids, user_table, item_table, fc_w, fc_b):
    uw = _sc_user_gather(user_ids, user_table)
    ir = _tc_item_gather(item_ids, item_table)
    return _tc_combine(uw, ir, fc_w, fc_b.reshape(1, 1))


# speedup vs baseline: 2.4579x; 2.4579x over previous
import functools

import jax
import jax.numpy as jnp
from jax import lax
from jax.experimental import pallas as pl
from jax.experimental.pallas import tpu as pltpu
from jax.experimental.pallas import tpu_sc as plsc

B = 16384
D = 64
L = 16
NC = 2
NS = 16
NW = NC * NS
BPW = B // NW
HALF = 256
NGROUP = HALF // L

_mesh = plsc.VectorSubcoreMesh(core_axis_name="c", subcore_axis_name="s")


@functools.partial(
    pl.kernel,
    mesh=_mesh,
    compiler_params=pltpu.CompilerParams(needs_layout_passes=False),
    out_type=jax.ShapeDtypeStruct((B,), jnp.float32),
    scratch_types=[
        pltpu.VMEM((BPW,), jnp.int32),
        pltpu.VMEM((BPW,), jnp.int32),
        pltpu.VMEM((HALF, D), jnp.float32),
        pltpu.VMEM((HALF, D), jnp.float32),
        pltpu.VMEM((D,), jnp.float32),
        pltpu.VMEM((L,), jnp.float32),
        pltpu.VMEM((BPW,), jnp.float32),
        pltpu.SemaphoreType.DMA,
        pltpu.SemaphoreType.DMA,
    ],
)
def _sc_kernel(uid_hbm, iid_hbm, ut_hbm, it_hbm, w_hbm, b_hbm, out_hbm,
               uidx_v, iidx_v, urows_v, irows_v, w_v, b_v, out_v,
               usem, isem):
    wid = lax.axis_index("s") * NC + lax.axis_index("c")
    base = wid * BPW

    pltpu.sync_copy(uid_hbm.at[pl.ds(base, BPW)], uidx_v)
    pltpu.sync_copy(iid_hbm.at[pl.ds(base, BPW)], iidx_v)
    pltpu.sync_copy(w_hbm, w_v)
    pltpu.sync_copy(b_hbm, b_v)

    wvecs = [w_v[pl.ds(j * L, L)] for j in range(D // L)]
    bvec = b_v[...]
    liota = lax.iota(jnp.int32, L)

    for h in range(2):
        hbase = h * HALF
        copies = []
        for k in range(HALF):
            if k % L == 0:
                uvec = uidx_v[pl.ds(hbase + k, L)]
                ivec = iidx_v[pl.ds(hbase + k, L)]
            u = uvec[k % L]
            i = ivec[k % L]
            copies.append(pltpu.async_copy(
                ut_hbm.at[u], urows_v.at[k], usem))
            copies.append(pltpu.async_copy(
                it_hbm.at[i], irows_v.at[k], isem))
        for cp in copies:
            cp.wait()

        def group_body(g, carry):
            r0 = g * L
            acc = bvec
            for rr in range(L):
                r = r0 + rr
                s = None
                for j in range(D // L):
                    t = (urows_v[r, pl.ds(j * L, L)]
                         * irows_v[r, pl.ds(j * L, L)] * wvecs[j])
                    s = t if s is None else s + t
                acc = jnp.where(liota == rr, acc + jnp.sum(s), acc)
            out_v[pl.ds(hbase + r0, L)] = acc
            return carry

        lax.fori_loop(0, NGROUP, group_body, 0, unroll=False)

    pltpu.sync_copy(out_v, out_hbm.at[pl.ds(base, BPW)])


def kernel(user_ids, item_ids, user_table, item_table, fc_w, fc_b):
    w = fc_w.reshape(D)
    b = jnp.broadcast_to(fc_b.reshape(1), (L,))
    return _sc_kernel(user_ids, item_ids, user_table, item_table, w, b)
